# Initial kernel scaffold; baseline (speedup 1.0000x reference)
import functools

import jax
import jax.numpy as jnp
from jax import lax
from jax.experimental import pallas as pl
from jax.experimental.pallas import tpu as pltpu
from jax.experimental.pallas import tpu_sc as plsc

_B = 4096
_NC = 26
_V = 1000
_ED = 64
_ND = 128
_BH = 256
_H = 2048
_DM = 1024
_D_IN = _ND + _NC * _ED + _BH

_SC_CORES = 2
_SC_SUBCORES = 16
_NW = _SC_CORES * _SC_SUBCORES


def _sc_gather(table_flat, flat_idx):
    n_rows = flat_idx.shape[0]
    ed = table_flat.shape[1]
    b_per_w = n_rows // _NW
    chunk = 832
    n_chunks = b_per_w // chunk

    mesh = plsc.VectorSubcoreMesh(
        core_axis_name="c", subcore_axis_name="s",
        num_cores=_SC_CORES, num_subcores=_SC_SUBCORES)

    @functools.partial(
        pl.kernel, mesh=mesh,
        out_type=jax.ShapeDtypeStruct((n_rows, ed), jnp.float32),
        scratch_types=[
            pltpu.VMEM((b_per_w,), jnp.int32),
            pltpu.VMEM((chunk, ed), jnp.float32),
            pltpu.SemaphoreType.DMA,
        ],
    )
    def gather_kernel(table_hbm, idx_hbm, out_hbm, idx_v, rows_v, sem):
        wid = lax.axis_index("s") * _SC_CORES + lax.axis_index("c")
        base = wid * b_per_w
        pltpu.sync_copy(idx_hbm.at[pl.ds(base, b_per_w)], idx_v)
        for j in range(n_chunks):
            pltpu.async_copy(
                table_hbm.at[idx_v.at[pl.ds(j * chunk, chunk)]], rows_v, sem
            ).wait()
            pltpu.sync_copy(rows_v, out_hbm.at[pl.ds(base + j * chunk, chunk)])

    return gather_kernel(table_flat, flat_idx)


def _mlp_body(num_r, cat_r, bank_r, g_r, b_r, w1_r, b1_r, w2_r, b2_r, out_r):
    x = jnp.concatenate([num_r[...], cat_r[...], bank_r[...]], axis=1)
    mu = jnp.mean(x, axis=1, keepdims=True)
    var = jnp.mean(jnp.square(x), axis=1, keepdims=True) - jnp.square(mu)
    xn = (x - mu) * jax.lax.rsqrt(var + 1e-5) * g_r[...] + b_r[...]
    h = jnp.dot(xn, w1_r[...], preferred_element_type=jnp.float32) + b1_r[...]
    h = 0.5 * h * (1.0 + jax.lax.erf(h * 0.7071067811865476))
    out_r[...] = jnp.dot(h, w2_r[...], preferred_element_type=jnp.float32) + b2_r[...]


def _mlp(num, cat_flat, bank, ln_g, ln_b, w1, b1, w2, b2, block_b=256):
    n_blocks = _B // block_b
    full = lambda shape: pl.BlockSpec(shape, lambda i: (0,) * len(shape))
    return pl.pallas_call(
        _mlp_body,
        grid=(n_blocks,),
        in_specs=[
            pl.BlockSpec((block_b, _ND), lambda i: (i, 0)),
            pl.BlockSpec((block_b, _NC * _ED), lambda i: (i, 0)),
            pl.BlockSpec((block_b, _BH), lambda i: (i, 0)),
            full((_D_IN,)),
            full((_D_IN,)),
            full((_D_IN, _H)),
            full((_H,)),
            full((_H, _DM)),
            full((_DM,)),
        ],
        out_specs=pl.BlockSpec((block_b, _DM), lambda i: (i, 0)),
        out_shape=jax.ShapeDtypeStruct((_B, _DM), jnp.float32),
    )(num, cat_flat, bank, ln_g, ln_b, w1, b1, w2, b2)


def kernel(trip_num_feat, trip_cat_feat, bank_context, emb_tables,
           ln_g, ln_b, W1, b1, W2, b2):
    table_flat = emb_tables.reshape(_NC * _V, _ED)
    offs = (jnp.arange(_NC, dtype=jnp.int32) * _V)[None, :]
    flat_idx = (trip_cat_feat + offs).reshape(-1)
    cat_rows = _sc_gather(table_flat, flat_idx)
    cat_flat = cat_rows.reshape(_B, _NC * _ED)
    return _mlp(trip_num_feat, cat_flat, bank_context, ln_g, ln_b, W1, b1, W2, b2)


# speedup vs baseline: 12.8360x; 12.8360x over previous
import functools

import jax
import jax.numpy as jnp
from jax import lax
from jax.experimental import pallas as pl
from jax.experimental.pallas import tpu as pltpu
from jax.experimental.pallas import tpu_sc as plsc

_B = 4096
_NC = 26
_V = 1000
_ED = 64
_ND = 128
_BH = 256
_H = 2048
_DM = 1024
_D_IN = _ND + _NC * _ED + _BH

_SC_CORES = 2
_SC_SUBCORES = 16
_NW = _SC_CORES * _SC_SUBCORES


def _sc_gather(table_flat, flat_idx):
    n_rows = flat_idx.shape[0]
    ed = table_flat.shape[1]
    b_per_w = n_rows // _NW
    chunk = 832
    n_chunks = b_per_w // chunk

    mesh = plsc.VectorSubcoreMesh(
        core_axis_name="c", subcore_axis_name="s",
        num_cores=_SC_CORES, num_subcores=_SC_SUBCORES)

    @functools.partial(
        pl.kernel, mesh=mesh,
        compiler_params=pltpu.CompilerParams(use_tc_tiling_on_sc=False),
        out_type=jax.ShapeDtypeStruct((n_rows, ed), jnp.float32),
        scratch_types=[
            pltpu.VMEM((b_per_w,), jnp.int32),
            pltpu.VMEM((chunk, ed), jnp.float32),
            pltpu.SemaphoreType.DMA,
        ],
    )
    def gather_kernel(table_hbm, idx_hbm, out_hbm, idx_v, rows_v, sem):
        wid = lax.axis_index("s") * _SC_CORES + lax.axis_index("c")
        base = wid * b_per_w
        pltpu.sync_copy(idx_hbm.at[pl.ds(base, b_per_w)], idx_v)
        for j in range(n_chunks):
            pltpu.async_copy(
                table_hbm.at[idx_v.at[pl.ds(j * chunk, chunk)]], rows_v, sem
            ).wait()
            pltpu.sync_copy(rows_v, out_hbm.at[pl.ds(base + j * chunk, chunk)])

    return gather_kernel(table_flat, flat_idx)


def _mlp_body(num_r, cat_r, bank_r, g_r, b_r, w1_r, b1_r, w2_r, b2_r, out_r):
    x = jnp.concatenate([num_r[...], cat_r[...], bank_r[...]], axis=1)
    mu = jnp.mean(x, axis=1, keepdims=True)
    var = jnp.mean(jnp.square(x), axis=1, keepdims=True) - jnp.square(mu)
    xn = (x - mu) * jax.lax.rsqrt(var + 1e-5) * g_r[...] + b_r[...]
    h = jnp.dot(xn, w1_r[...], preferred_element_type=jnp.float32) + b1_r[...]
    h = 0.5 * h * (1.0 + jax.lax.erf(h * 0.7071067811865476))
    out_r[...] = jnp.dot(h, w2_r[...], preferred_element_type=jnp.float32) + b2_r[...]


def _mlp(num, cat_flat, bank, ln_g, ln_b, w1, b1, w2, b2, block_b=256):
    n_blocks = _B // block_b
    full = lambda shape: pl.BlockSpec(shape, lambda i: (0,) * len(shape))
    return pl.pallas_call(
        _mlp_body,
        grid=(n_blocks,),
        in_specs=[
            pl.BlockSpec((block_b, _ND), lambda i: (i, 0)),
            pl.BlockSpec((block_b, _NC * _ED), lambda i: (i, 0)),
            pl.BlockSpec((block_b, _BH), lambda i: (i, 0)),
            full((_D_IN,)),
            full((_D_IN,)),
            full((_D_IN, _H)),
            full((_H,)),
            full((_H, _DM)),
            full((_DM,)),
        ],
        out_specs=pl.BlockSpec((block_b, _DM), lambda i: (i, 0)),
        out_shape=jax.ShapeDtypeStruct((_B, _DM), jnp.float32),
    )(num, cat_flat, bank, ln_g, ln_b, w1, b1, w2, b2)


def kernel(trip_num_feat, trip_cat_feat, bank_context, emb_tables,
           ln_g, ln_b, W1, b1, W2, b2):
    table_flat = emb_tables.reshape(_NC * _V, _ED)
    offs = (jnp.arange(_NC, dtype=jnp.int32) * _V)[None, :]
    flat_idx = (trip_cat_feat + offs).reshape(-1)
    cat_rows = _sc_gather(table_flat, flat_idx)
    cat_flat = cat_rows.reshape(_B, _NC * _ED)
    return _mlp(trip_num_feat, cat_flat, bank_context, ln_g, ln_b, W1, b1, W2, b2)
